# baseline (speedup 1.0000x reference)
import functools
import math

import jax
import jax.numpy as jnp
from jax import lax
from jax.experimental import pallas as pl
from jax.experimental.pallas import tpu as pltpu
from jax.experimental.pallas import tpu_sc as plsc

N = 10000
D = 128
DH = D // 2
E = 320000

NC = 2
NS = 16
NW = NC * NS
LANES = 128
ROWS_TOTAL = E // LANES
ROWS_PER_W = ROWS_TOTAL // NW
REM_ROW0 = NW * ROWS_PER_W
K = 3
CHUNKS = ROWS_PER_W // K
NPAIR = CHUNKS // 2
N_PAD = 10240
STRIPE = N_PAD // NS


def _sc_segment_sum(src2d, dst2d, x2d):
    mesh = plsc.VectorSubcoreMesh(core_axis_name="c", subcore_axis_name="s")

    @functools.partial(
        pl.kernel,
        out_type=[
            jax.ShapeDtypeStruct((NC, N_PAD, DH), jnp.float32),
            jax.ShapeDtypeStruct((NC, N_PAD, DH), jnp.float32),
            jax.ShapeDtypeStruct((NC, N_PAD // 16, 16), jnp.float32),
        ],
        mesh=mesh,
        scratch_types=[
            pltpu.VMEM((K * LANES,), jnp.int32),
            pltpu.VMEM((K, LANES), jnp.int32),
            pltpu.VMEM((K * LANES,), jnp.int32),
            pltpu.VMEM((K, LANES), jnp.int32),
            pltpu.VMEM((K * LANES, DH), jnp.float32),
            pltpu.VMEM((K * LANES, DH), jnp.float32),
            pltpu.VMEM((STRIPE, 16), jnp.float32),
            pltpu.VMEM((STRIPE,), jnp.int32),
            pltpu.VMEM((STRIPE // 4, 16), jnp.float32),
            pltpu.VMEM((STRIPE // 4, DH), jnp.float32),
            pltpu.VMEM_SHARED((N_PAD, DH), jnp.float32),
            pltpu.VMEM_SHARED((N_PAD // 16, 16), jnp.float32),
            pltpu.SemaphoreType.DMA,
            pltpu.SemaphoreType.DMA,
        ],
        compiler_params=pltpu.CompilerParams(use_tc_tiling_on_sc=False, needs_layout_passes=False),
    )
    def body(src_hbm, dst_hbm, x_hbm, agg0_out, agg1_out, cnt_out,
             gixA, dstA, gixB, dstB, rowsA, rowsB,
             cntl_v, iota_v, zc_v, za_v, agg_sh, cnt_sh, sem_g, sem_s):
        cid = lax.axis_index("c")
        sid = lax.axis_index("s")
        wid = sid * NC + cid

        zero16 = jnp.zeros((16,), jnp.float32)
        one16 = jnp.ones((16,), jnp.float32)

        def init_zc(i, _):
            zc_v[i, pl.ds(0, 16)] = zero16
            for c in range(DH // 16):
                za_v[i, pl.ds(c * 16, 16)] = zero16
            return 0

        def init_cntl(i, _):
            cntl_v[i, pl.ds(0, 16)] = zero16
            return 0

        def init_iota(i, _):
            iota_v[pl.ds(i * 16, 16)] = lax.iota(jnp.int32, 16) + i * 16
            return 0

        def load_idx(row0, gix_v, dst_v, p):
            pltpu.sync_copy(src_hbm.at[pl.ds(row0 * LANES, K * LANES)], gix_v)
            pltpu.sync_copy(dst_hbm.at[pl.ds(row0, K)], dst_v)
            for c in range(K * LANES // 16):
                s16 = gix_v[pl.ds(c * 16, 16)]
                gix_v[pl.ds(c * 16, 16)] = s16 * 2 + p

        def fire_gather(gix_v, rows_v):
            return pltpu.async_copy(x_hbm.at[gix_v], rows_v, sem_g)

        def fire_scatters(rows_v, dst_v, p):
            return [
                pltpu.async_copy(rows_v.at[pl.ds(b * LANES, LANES)],
                                 agg_sh.at[dst_v.at[b]], sem_s, add=True)
                for b in range(K)
            ]

        def count_local(dst_v, nrows):
            for r in range(nrows):
                for c in range(LANES // 16):
                    d16 = dst_v[r, pl.ds(c * 16, 16)]
                    plsc.addupdate_scatter(
                        cntl_v,
                        [lax.shift_right_logical(d16, 4),
                         lax.bitwise_and(d16, 15)],
                        one16)

        def wait_all(descs):
            for d_ in descs:
                d_.wait()

        for p, agg_out in enumerate((agg0_out, agg1_out)):
            base = wid * ROWS_PER_W

            load_idx(base, gixA, dstA, p)
            g0 = fire_gather(gixA, rowsA)

            if p == 0:
                lax.fori_loop(0, STRIPE // 4, init_zc, 0)
                lax.fori_loop(0, STRIPE, init_cntl, 0)
                lax.fori_loop(0, STRIPE // 16, init_iota, 0)

            for z in range(4):
                zoff = sid * STRIPE + z * (STRIPE // 4)
                pltpu.sync_copy(za_v, agg_sh.at[pl.ds(zoff, STRIPE // 4)])
            if p == 0:
                pltpu.sync_copy(
                    zc_v.at[pl.ds(0, STRIPE // 16)],
                    cnt_sh.at[pl.ds(sid * (STRIPE // 16), STRIPE // 16)])
            plsc.subcore_barrier()
            g0.wait()

            def pair(q, _):
                rowA = base + (2 * q) * K
                rowB = rowA + K
                sA = fire_scatters(rowsA, dstA, p)
                if p == 0:
                    count_local(dstA, K)
                load_idx(rowB, gixB, dstB, p)
                gB = fire_gather(gixB, rowsB)
                wait_all(sA)
                gB.wait()
                sB = fire_scatters(rowsB, dstB, p)
                if p == 0:
                    count_local(dstB, K)

                @pl.when(q + 1 < NPAIR)
                def _():
                    load_idx(rowB + K, gixA, dstA, p)
                    fire_gather(gixA, rowsA).wait()

                wait_all(sB)
                return 0

            lax.fori_loop(0, NPAIR, pair, 0)

            @pl.when(wid < 4)
            def _():
                row0 = REM_ROW0 + wid
                pltpu.sync_copy(src_hbm.at[pl.ds(row0 * LANES, LANES)],
                                gixB.at[pl.ds(0, LANES)])
                pltpu.sync_copy(dst_hbm.at[pl.ds(row0, 1)],
                                dstB.at[pl.ds(0, 1)])
                for c in range(LANES // 16):
                    s16 = gixB[pl.ds(c * 16, 16)]
                    gixB[pl.ds(c * 16, 16)] = s16 * 2 + p
                g = pltpu.async_copy(x_hbm.at[gixB.at[pl.ds(0, LANES)]],
                                     rowsB.at[pl.ds(0, LANES)], sem_g)
                g.wait()
                pltpu.sync_copy(rowsB.at[pl.ds(0, LANES)],
                                agg_sh.at[dstB.at[0]], add=True)
                if p == 0:
                    count_local(dstB, 1)

            if p == 0:
                pltpu.sync_copy(cntl_v, cnt_sh.at[iota_v], add=True)
            plsc.subcore_barrier()
            wb = [pltpu.async_copy(agg_sh.at[pl.ds(sid * STRIPE, STRIPE)],
                                   agg_out.at[cid, pl.ds(sid * STRIPE, STRIPE)],
                                   sem_g)]
            if p == 0:
                wb.append(
                    pltpu.async_copy(
                        cnt_sh.at[pl.ds(sid * (STRIPE // 16), STRIPE // 16)],
                        cnt_out.at[cid, pl.ds(sid * (STRIPE // 16),
                                              STRIPE // 16)],
                        sem_g))
            wait_all(wb)

    return body(src2d, dst2d, x2d)


BLK = 2000
GRID = N // BLK


def _tc_dense(x, agg0, agg1, cnt2, W_l, b_l, W_r, ln_gamma, ln_beta):
    inv_sqrt2 = 1.0 / math.sqrt(2.0)

    def body(x_ref, a0_ref, a1_ref, c_ref, wl_ref, bl_ref, wr_ref, g_ref,
             be_ref, o_ref):
        inv = 1.0 / jnp.maximum(c_ref[0] + c_ref[1], 1.0)
        m_lo = (a0_ref[0] + a0_ref[1]) * inv
        m_hi = (a1_ref[0] + a1_ref[1]) * inv
        x_b = x_ref[...]
        h = (jnp.dot(m_lo, wl_ref[:DH, :], preferred_element_type=jnp.float32)
             + jnp.dot(m_hi, wl_ref[DH:, :], preferred_element_type=jnp.float32)
             + jnp.dot(x_b, wr_ref[...], preferred_element_type=jnp.float32)
             + bl_ref[...])
        mu = jnp.mean(h, axis=-1, keepdims=True)
        d = h - mu
        var = jnp.mean(d * d, axis=-1, keepdims=True)
        hn = d * lax.rsqrt(var + 1e-5) * g_ref[...] + be_ref[...]
        act = hn * 0.5 * (1.0 + lax.erf(hn * inv_sqrt2))
        o_ref[...] = act + x_b

    return pl.pallas_call(
        body,
        grid=(GRID,),
        in_specs=[
            pl.BlockSpec((BLK, D), lambda i: (i, 0)),
            pl.BlockSpec((NC, BLK, DH), lambda i: (0, i, 0)),
            pl.BlockSpec((NC, BLK, DH), lambda i: (0, i, 0)),
            pl.BlockSpec((NC, BLK, 1), lambda i: (0, i, 0)),
            pl.BlockSpec((D, D), lambda i: (0, 0)),
            pl.BlockSpec((1, D), lambda i: (0, 0)),
            pl.BlockSpec((D, D), lambda i: (0, 0)),
            pl.BlockSpec((1, D), lambda i: (0, 0)),
            pl.BlockSpec((1, D), lambda i: (0, 0)),
        ],
        out_specs=pl.BlockSpec((BLK, D), lambda i: (i, 0)),
        out_shape=jax.ShapeDtypeStruct((N, D), jnp.float32),
    )(x, agg0, agg1, cnt2, W_l, b_l.reshape(1, D), W_r,
      ln_gamma.reshape(1, D), ln_beta.reshape(1, D))


def kernel(x, edge_index, W_l, b_l, W_r, ln_gamma, ln_beta):
    src2d = edge_index[0]
    dst2d = edge_index[1].reshape(ROWS_TOTAL, LANES)
    x2d = x.reshape(2 * N, DH)
    agg0, agg1, cnth = _sc_segment_sum(src2d, dst2d, x2d)
    cnt2 = cnth.reshape(NC, N_PAD, 1)
    return _tc_dense(x, agg0, agg1, cnt2, W_l, b_l, W_r, ln_gamma, ln_beta)


# speedup vs baseline: 1.0350x; 1.0350x over previous
import functools
import math

import jax
import jax.numpy as jnp
from jax import lax
from jax.experimental import pallas as pl
from jax.experimental.pallas import tpu as pltpu
from jax.experimental.pallas import tpu_sc as plsc

N = 10000
D = 128
DH = D // 2
E = 320000

NC = 2
NS = 16
NW = NC * NS
LANES = 128
ROWS_TOTAL = E // LANES
ROWS_PER_W = ROWS_TOTAL // NW
REM_ROW0 = NW * ROWS_PER_W
K = 3
CHUNKS = ROWS_PER_W // K
NPAIR = CHUNKS // 2
N_PAD = 10240
STRIPE = N_PAD // NS


def _sc_segment_sum(src2d, dst2d, x2d):
    mesh = plsc.VectorSubcoreMesh(core_axis_name="c", subcore_axis_name="s")

    @functools.partial(
        pl.kernel,
        out_type=[
            jax.ShapeDtypeStruct((NC, N_PAD, DH), jnp.float32),
            jax.ShapeDtypeStruct((NC, N_PAD, DH), jnp.float32),
            jax.ShapeDtypeStruct((NC, N_PAD, 16), jnp.float32),
        ],
        mesh=mesh,
        scratch_types=[
            pltpu.VMEM((K * LANES,), jnp.int32),
            pltpu.VMEM((K, LANES), jnp.int32),
            pltpu.VMEM((K * LANES,), jnp.int32),
            pltpu.VMEM((K, LANES), jnp.int32),
            pltpu.VMEM((K * LANES, DH), jnp.float32),
            pltpu.VMEM((K * LANES, DH), jnp.float32),
            pltpu.VMEM((LANES, 16), jnp.float32),
            pltpu.VMEM((STRIPE // 4, 16), jnp.float32),
            pltpu.VMEM((STRIPE // 4, DH), jnp.float32),
            pltpu.VMEM_SHARED((N_PAD, DH), jnp.float32),
            pltpu.VMEM_SHARED((N_PAD, 16), jnp.float32),
            pltpu.SemaphoreType.DMA,
            pltpu.SemaphoreType.DMA,
        ],
        compiler_params=pltpu.CompilerParams(use_tc_tiling_on_sc=False),
    )
    def body(src_hbm, dst_hbm, x_hbm, agg0_out, agg1_out, cnt_out,
             gixA, dstA, gixB, dstB, rowsA, rowsB,
             ones_v, zc_v, za_v, agg_sh, cnt_sh, sem_g, sem_s):
        cid = lax.axis_index("c")
        sid = lax.axis_index("s")
        wid = sid * NC + cid

        zero16 = jnp.zeros((16,), jnp.float32)
        one16 = jnp.ones((16,), jnp.float32)

        def init_zc(i, _):
            zc_v[i, pl.ds(0, 16)] = zero16
            for c in range(DH // 16):
                za_v[i, pl.ds(c * 16, 16)] = zero16
            return 0

        def init_ones(i, _):
            ones_v[i, pl.ds(0, 16)] = one16
            return 0

        lax.fori_loop(0, LANES, init_ones, 0)

        def load_idx(row0, gix_v, dst_v, p):
            pltpu.sync_copy(src_hbm.at[pl.ds(row0 * LANES, K * LANES)], gix_v)
            pltpu.sync_copy(dst_hbm.at[pl.ds(row0, K)], dst_v)
            for c in range(K * LANES // 16):
                s16 = gix_v[pl.ds(c * 16, 16)]
                gix_v[pl.ds(c * 16, 16)] = s16 * 2 + p

        def fire_gather(gix_v, rows_v):
            return pltpu.async_copy(x_hbm.at[gix_v], rows_v, sem_g)

        def fire_scatters(rows_v, dst_v, p):
            out = []
            for b in range(K):
                out.append(
                    pltpu.async_copy(rows_v.at[pl.ds(b * LANES, LANES)],
                                     agg_sh.at[dst_v.at[b]], sem_s, add=True))
                if p == 0:
                    out.append(
                        pltpu.async_copy(ones_v, cnt_sh.at[dst_v.at[b]],
                                         sem_s, add=True))
            return out

        def wait_all(descs):
            for d_ in descs:
                d_.wait()

        for p, agg_out in enumerate((agg0_out, agg1_out)):
            base = wid * ROWS_PER_W

            load_idx(base, gixA, dstA, p)
            g0 = fire_gather(gixA, rowsA)

            if p == 0:
                lax.fori_loop(0, STRIPE // 4, init_zc, 0)
                lax.fori_loop(0, LANES, init_ones, 0)

            for z in range(4):
                zoff = sid * STRIPE + z * (STRIPE // 4)
                pltpu.sync_copy(za_v, agg_sh.at[pl.ds(zoff, STRIPE // 4)])
                if p == 0:
                    pltpu.sync_copy(zc_v, cnt_sh.at[pl.ds(zoff, STRIPE // 4)])
            plsc.subcore_barrier()
            g0.wait()

            def pair(q, _):
                rowA = base + (2 * q) * K
                rowB = rowA + K
                sA = fire_scatters(rowsA, dstA, p)
                load_idx(rowB, gixB, dstB, p)
                gB = fire_gather(gixB, rowsB)
                wait_all(sA)
                gB.wait()
                sB = fire_scatters(rowsB, dstB, p)

                @pl.when(q + 1 < NPAIR)
                def _():
                    load_idx(rowB + K, gixA, dstA, p)
                    fire_gather(gixA, rowsA).wait()

                wait_all(sB)
                return 0

            lax.fori_loop(0, NPAIR, pair, 0)

            @pl.when(wid < 4)
            def _():
                row0 = REM_ROW0 + wid
                pltpu.sync_copy(src_hbm.at[pl.ds(row0 * LANES, LANES)],
                                gixB.at[pl.ds(0, LANES)])
                pltpu.sync_copy(dst_hbm.at[pl.ds(row0, 1)],
                                dstB.at[pl.ds(0, 1)])
                for c in range(LANES // 16):
                    s16 = gixB[pl.ds(c * 16, 16)]
                    gixB[pl.ds(c * 16, 16)] = s16 * 2 + p
                g = pltpu.async_copy(x_hbm.at[gixB.at[pl.ds(0, LANES)]],
                                     rowsB.at[pl.ds(0, LANES)], sem_g)
                g.wait()
                s = [pltpu.async_copy(rowsB.at[pl.ds(0, LANES)],
                                      agg_sh.at[dstB.at[0]], sem_s, add=True)]
                if p == 0:
                    s.append(pltpu.async_copy(ones_v, cnt_sh.at[dstB.at[0]],
                                              sem_s, add=True))
                wait_all(s)

            plsc.subcore_barrier()
            wb = [pltpu.async_copy(agg_sh.at[pl.ds(sid * STRIPE, STRIPE)],
                                   agg_out.at[cid, pl.ds(sid * STRIPE, STRIPE)],
                                   sem_g)]
            if p == 0:
                wb.append(
                    pltpu.async_copy(cnt_sh.at[pl.ds(sid * STRIPE, STRIPE)],
                                     cnt_out.at[cid, pl.ds(sid * STRIPE, STRIPE)],
                                     sem_g))
            wait_all(wb)

    return body(src2d, dst2d, x2d)


BLK = 2000
GRID = N // BLK


def _tc_dense(x, agg0, agg1, cnt2, W_l, b_l, W_r, ln_gamma, ln_beta):
    inv_sqrt2 = 1.0 / math.sqrt(2.0)

    def body(x_ref, a0_ref, a1_ref, c_ref, wl_ref, bl_ref, wr_ref, g_ref,
             be_ref, o_ref):
        inv = 1.0 / jnp.maximum(c_ref[0, :, :1] + c_ref[1, :, :1], 1.0)
        m_lo = (a0_ref[0] + a0_ref[1]) * inv
        m_hi = (a1_ref[0] + a1_ref[1]) * inv
        x_b = x_ref[...]
        h = (jnp.dot(m_lo, wl_ref[:DH, :], preferred_element_type=jnp.float32)
             + jnp.dot(m_hi, wl_ref[DH:, :], preferred_element_type=jnp.float32)
             + jnp.dot(x_b, wr_ref[...], preferred_element_type=jnp.float32)
             + bl_ref[...])
        mu = jnp.mean(h, axis=-1, keepdims=True)
        d = h - mu
        var = jnp.mean(d * d, axis=-1, keepdims=True)
        hn = d * lax.rsqrt(var + 1e-5) * g_ref[...] + be_ref[...]
        act = hn * 0.5 * (1.0 + lax.erf(hn * inv_sqrt2))
        o_ref[...] = act + x_b

    return pl.pallas_call(
        body,
        grid=(GRID,),
        in_specs=[
            pl.BlockSpec((BLK, D), lambda i: (i, 0)),
            pl.BlockSpec((NC, BLK, DH), lambda i: (0, i, 0)),
            pl.BlockSpec((NC, BLK, DH), lambda i: (0, i, 0)),
            pl.BlockSpec((NC, BLK, 16), lambda i: (0, i, 0)),
            pl.BlockSpec((D, D), lambda i: (0, 0)),
            pl.BlockSpec((1, D), lambda i: (0, 0)),
            pl.BlockSpec((D, D), lambda i: (0, 0)),
            pl.BlockSpec((1, D), lambda i: (0, 0)),
            pl.BlockSpec((1, D), lambda i: (0, 0)),
        ],
        out_specs=pl.BlockSpec((BLK, D), lambda i: (i, 0)),
        out_shape=jax.ShapeDtypeStruct((N, D), jnp.float32),
    )(x, agg0, agg1, cnt2, W_l, b_l.reshape(1, D), W_r,
      ln_gamma.reshape(1, D), ln_beta.reshape(1, D))


def kernel(x, edge_index, W_l, b_l, W_r, ln_gamma, ln_beta):
    src2d = edge_index[0]
    dst2d = edge_index[1].reshape(ROWS_TOTAL, LANES)
    x2d = x.reshape(2 * N, DH)
    agg0, agg1, cnt2 = _sc_segment_sum(src2d, dst2d, x2d)
    return _tc_dense(x, agg0, agg1, cnt2, W_l, b_l, W_r, ln_gamma, ln_beta)
